# baseline (speedup 1.0000x reference)
import functools

import jax
import jax.numpy as jnp
from jax import lax
from jax.experimental import pallas as pl
from jax.experimental.pallas import tpu as pltpu
from jax.experimental.pallas import tpu_sc as plsc

N_NODES = 10000
N_EDGES = 320000
F = 128

NC = 2
NS = 16
L = 16

N_PAD = 10240
CHUNK = 48
BLK = 8
CPT = 224
NBLK = CPT // BLK
E_PAD = NC * NS * CPT * CHUNK
ROWS_PER_TILE = N_PAD // NS
NBUF = 4
ZB = 40
H1ROWS = 2816
H1R = H1ROWS // NS


_CMAP = [32 * q + 2 * k + u for q in range(F // 32)
         for u in range(2) for k in range(L)]


def _rsqrt_newton(d):
    s = d
    for _ in range(14):
        s = 0.5 * (s + d / s)
    return 1.0 / s


def _bcast(ref, i):
    return plsc.load_gather(ref, [jnp.full((L,), i, jnp.int32)])


def _sc_body(src_hbm, dst_hbm, ew_hbm, feat_hbm, src128_hbm,
             acc_out, dh_out,
             acc_sh, hist_s_sh, hist_d_sh,
             sblk, dblk, eblk, sblk1, ones_t, ones_c, norm_s_t, nd_t,
             gbuf, sbuf, w_t, gsem, ssem, hsem):
    core = lax.axis_index("c")
    sub = lax.axis_index("s")
    zero16 = jnp.zeros((L,), jnp.float32)
    one16 = jnp.ones((L,), jnp.float32)
    row0 = sub * ROWS_PER_TILE

    def _zrow(r, _):
        for q in range(F // L):
            sbuf.at[0, r][pl.ds(q * L, L)] = zero16
        return _
    lax.fori_loop(0, CHUNK, _zrow, None)
    for q in range(128 // L):
        ones_t[pl.ds(q * L, L)] = one16
    for q in range(CHUNK // L):
        ones_c[pl.ds(q * L, L)] = one16
    for q in range(ROWS_PER_TILE // L):
        nd_t[pl.ds(q * L, L)] = zero16

    for k in range(ROWS_PER_TILE // ZB):
        pltpu.async_copy(sbuf.at[0, pl.ds(0, ZB)],
                         acc_sh.at[pl.ds(row0 + k * ZB, ZB)], hsem)
    pltpu.sync_copy(nd_t, hist_s_sh.at[pl.ds(row0, ROWS_PER_TILE)])
    pltpu.sync_copy(nd_t, hist_d_sh.at[pl.ds(row0, ROWS_PER_TILE)])
    for k in range(ROWS_PER_TILE // ZB):
        pltpu.make_async_copy(sbuf.at[0, pl.ds(0, ZB)],
                              acc_sh.at[pl.ds(row0 + k * ZB, ZB)],
                              hsem).wait()
    plsc.subcore_barrier()

    def _h1(k, _):
        p = k % 2
        base = sub * H1R + k * BLK
        pltpu.sync_copy(src128_hbm.at[pl.ds(base, BLK)], sblk1.at[p])

        @pl.when(k >= 1)
        def _drain():
            for _i in range(BLK):
                pltpu.make_async_copy(ones_t, hist_s_sh.at[sblk1.at[0, 0]],
                                      hsem).wait()

        for i in range(BLK):
            pltpu.async_copy(ones_t, hist_s_sh.at[sblk1.at[p, i]],
                             hsem, add=True)
        return _
    lax.fori_loop(0, H1R // BLK, _h1, None)
    for _i in range(BLK):
        pltpu.make_async_copy(ones_t, hist_s_sh.at[sblk1.at[0, 0]],
                              hsem).wait()
    plsc.subcore_barrier()

    def _norm(ref):
        def _n(g, _):
            d = jnp.maximum(ref[pl.ds(g * L, L)], 1.0)
            ref[pl.ds(g * L, L)] = _rsqrt_newton(d)
            return _
        lax.fori_loop(0, ROWS_PER_TILE // L, _n, None)

    pltpu.sync_copy(hist_s_sh.at[pl.ds(row0, ROWS_PER_TILE)], nd_t)
    _norm(nd_t)
    pltpu.sync_copy(nd_t, hist_s_sh.at[pl.ds(row0, ROWS_PER_TILE)])
    plsc.subcore_barrier()
    pltpu.sync_copy(hist_s_sh, norm_s_t)

    base3 = (core * NS + sub) * CPT

    def _g_issue(pp, r, buf):
        pltpu.async_copy(feat_hbm.at[sblk.at[pp, r]], gbuf.at[buf],
                         gsem.at[buf])

    def _g_wait(buf):
        pltpu.make_async_copy(feat_hbm.at[sblk.at[0, 0]], gbuf.at[buf],
                              gsem.at[buf]).wait()

    def _s_issue(pp, r, sb):
        pltpu.async_copy(sbuf.at[sb], acc_sh.at[dblk.at[pp, r]],
                         ssem.at[sb], add=True)
        pltpu.async_copy(ones_c, hist_d_sh.at[dblk.at[pp, r]],
                         ssem.at[sb], add=True)

    def _s_wait(sb):
        pltpu.make_async_copy(sbuf.at[sb], acc_sh.at[dblk.at[0, 0]],
                              ssem.at[sb]).wait()
        pltpu.make_async_copy(ones_c, hist_d_sh.at[dblk.at[0, 0]],
                              ssem.at[sb]).wait()

    pltpu.sync_copy(src_hbm.at[pl.ds(base3, BLK)], sblk.at[0])
    pltpu.sync_copy(dst_hbm.at[pl.ds(base3, BLK)], dblk.at[0])
    pltpu.sync_copy(ew_hbm.at[pl.ds(base3, BLK)], eblk.at[0])
    pltpu.sync_copy(src_hbm.at[pl.ds(base3 + BLK, BLK)], sblk.at[1])
    pltpu.sync_copy(dst_hbm.at[pl.ds(base3 + BLK, BLK)], dblk.at[1])
    pltpu.sync_copy(ew_hbm.at[pl.ds(base3 + BLK, BLK)], eblk.at[1])
    for i in range(NBUF - 1):
        _g_issue(0, i, i)

    hi_mask = jnp.full((L,), -65536, jnp.int32)

    def _b3(b, _):
        p = b % 2
        for i in range(BLK):
            buf = i % NBUF
            sb = i % 2
            _g_wait(buf)
            for q in range(CHUNK // L):
                sv = sblk.at[p, i][pl.ds(q * L, L)]
                ev = eblk.at[p, i][pl.ds(q * L, L)]
                w_t[pl.ds(q * L, L)] = ev * plsc.load_gather(norm_s_t, [sv])

            def _scale(r, _c):
                w = _bcast(w_t, r)
                gv = gbuf.at[buf, r]
                sv_ = sbuf.at[sb, r]
                for q in range(F // 32):
                    bi = gv[pl.ds(L * q, L)]
                    lo = plsc.bitcast(bi << 16, jnp.float32)
                    hi = plsc.bitcast(bi & hi_mask, jnp.float32)
                    sv_[pl.ds(32 * q, L)] = lo * w
                    sv_[pl.ds(32 * q + L, L)] = hi * w
                return _c
            lax.fori_loop(0, CHUNK, _scale, None)
            _s_issue(p, i, sb)

            psb = (i - 1) % 2
            if i == 0:
                @pl.when(b > 0)
                def _w0():
                    _s_wait(psb)
            else:
                _s_wait(psb)
            if i == 5:
                @pl.when(b + 1 < NBLK)
                def _pref():
                    nb = base3 + (b + 1) * BLK
                    pltpu.sync_copy(src_hbm.at[pl.ds(nb, BLK)],
                                    sblk.at[1 - p])
                    pltpu.sync_copy(dst_hbm.at[pl.ds(nb, BLK)],
                                    dblk.at[1 - p])
                    pltpu.sync_copy(ew_hbm.at[pl.ds(nb, BLK)],
                                    eblk.at[1 - p])
            nxt = b * BLK + i + NBUF - 1
            gb = (i + NBUF - 1) % NBUF

            @pl.when(nxt < CPT)
            def _gi():
                if i + NBUF - 1 < BLK:
                    _g_issue(p, i + NBUF - 1, gb)
                else:
                    _g_issue(1 - p, i + NBUF - 1 - BLK, gb)
        return _
    lax.fori_loop(0, NBLK, _b3, None)
    _s_wait((CPT - 1) % 2)
    plsc.subcore_barrier()

    pltpu.sync_copy(acc_sh.at[pl.ds(row0, ROWS_PER_TILE)],
                    acc_out.at[core, pl.ds(row0, ROWS_PER_TILE)])
    pltpu.sync_copy(hist_d_sh.at[pl.ds(row0, ROWS_PER_TILE)],
                    dh_out.at[core, pl.ds(row0, ROWS_PER_TILE)])


def _sc_aggregate(src_p, dst_p, ew_p, feat_p, src128):
    mesh = plsc.VectorSubcoreMesh(core_axis_name="c", subcore_axis_name="s")
    return pl.kernel(
        _sc_body,
        out_type=[
            jax.ShapeDtypeStruct((NC, N_PAD, F), jnp.float32),
            jax.ShapeDtypeStruct((NC, N_PAD), jnp.float32),
        ],
        mesh=mesh,
        compiler_params=pltpu.CompilerParams(needs_layout_passes=False,
                                             use_tc_tiling_on_sc=False),
        scratch_types=[
            pltpu.VMEM_SHARED((N_PAD, F), jnp.float32),
            pltpu.VMEM_SHARED((N_PAD,), jnp.float32),
            pltpu.VMEM_SHARED((N_PAD,), jnp.float32),
            pltpu.VMEM((2, BLK, CHUNK), jnp.int32),
            pltpu.VMEM((2, BLK, CHUNK), jnp.int32),
            pltpu.VMEM((2, BLK, CHUNK), jnp.float32),
            pltpu.VMEM((2, BLK, 128), jnp.int32),
            pltpu.VMEM((128,), jnp.float32),
            pltpu.VMEM((CHUNK,), jnp.float32),
            pltpu.VMEM((N_PAD,), jnp.float32),
            pltpu.VMEM((ROWS_PER_TILE,), jnp.float32),
            pltpu.VMEM((NBUF, CHUNK, F // 2), jnp.int32),
            pltpu.VMEM((2, CHUNK, F), jnp.float32),
            pltpu.VMEM((CHUNK,), jnp.float32),
            pltpu.SemaphoreType.DMA((NBUF,)),
            pltpu.SemaphoreType.DMA((NBUF,)),
            pltpu.SemaphoreType.DMA,
        ],
    )(src_p, dst_p, ew_p, feat_p, src128)


def _tc_body(acc_ref, dh_ref, w_ref, b_ref, out_ref):
    p = acc_ref[0] + acc_ref[1]
    y = jnp.dot(p, w_ref[...], preferred_element_type=jnp.float32)
    d = jnp.maximum(dh_ref[0] + dh_ref[1], 1.0)
    out_ref[...] = y * lax.rsqrt(d) + b_ref[...]


def _tc_matmul(acc, dh, W, b2):
    blk = 1000
    grid = (N_NODES // blk,)
    return pl.pallas_call(
        _tc_body,
        grid=grid,
        in_specs=[
            pl.BlockSpec((NC, blk, F), lambda i: (0, i, 0)),
            pl.BlockSpec((NC, blk, 1), lambda i: (0, i, 0)),
            pl.BlockSpec((F, F), lambda i: (0, 0)),
            pl.BlockSpec((1, F), lambda i: (0, 0)),
        ],
        out_specs=pl.BlockSpec((blk, F), lambda i: (i, 0)),
        out_shape=jax.ShapeDtypeStruct((N_NODES, F), jnp.float32),
    )(acc, dh, W, b2)


@jax.jit
def kernel(feat, edge_index, edge_weight, W, b):
    src = edge_index[0].astype(jnp.int32)
    dst = edge_index[1].astype(jnp.int32)
    e = src.shape[0]
    npad = E_PAD - e
    pad_idx = (jnp.arange(npad, dtype=jnp.int32) % (N_PAD - N_NODES)) + N_NODES
    src_flat = jnp.concatenate([src, pad_idx])
    src_p = src_flat.reshape(E_PAD // CHUNK, CHUNK)
    npad1 = H1ROWS * 128 - E_PAD
    pad1 = (jnp.arange(npad1, dtype=jnp.int32) % (N_PAD - N_NODES)) + N_NODES
    src128 = jnp.concatenate([src_flat, pad1]).reshape(H1ROWS, 128)
    dst_p = jnp.concatenate([dst, pad_idx]).reshape(E_PAD // CHUNK, CHUNK)
    ew_p = jnp.concatenate(
        [edge_weight, jnp.zeros((npad,), jnp.float32)]
    ).reshape(E_PAD // CHUNK, CHUNK)
    feat_p = jnp.pad(feat, ((0, N_PAD - feat.shape[0]), (0, 0)))
    feat_bf = lax.bitcast_convert_type(
        feat_p.astype(jnp.bfloat16).reshape(N_PAD, F // 2, 2), jnp.int32)
    acc, dh = _sc_aggregate(src_p, dst_p, ew_p, feat_bf, src128)
    cmap = jnp.asarray(_CMAP, dtype=jnp.int32)
    return _tc_matmul(acc, dh.reshape(NC, N_PAD, 1), W[cmap], b.reshape(1, F))


# speedup vs baseline: 1.6443x; 1.6443x over previous
import functools

import jax
import jax.numpy as jnp
from jax import lax
from jax.experimental import pallas as pl
from jax.experimental.pallas import tpu as pltpu
from jax.experimental.pallas import tpu_sc as plsc

N_NODES = 10000
N_EDGES = 320000
F = 128

NC = 2
NS = 16
L = 16

N_PAD = 10240
CHUNK = 48
BLK = 8
CPT = 224
NBLK = CPT // BLK
E_PAD = NC * NS * CPT * CHUNK
ROWS_PER_TILE = N_PAD // NS
NBUF = 4
ZB = 40
H1ROWS = 2816
H1R = H1ROWS // NS


def _rsqrt_newton(d):
    s = d
    for _ in range(14):
        s = 0.5 * (s + d / s)
    return 1.0 / s


def _bcast(ref, i):
    return plsc.load_gather(ref, [jnp.full((L,), i, jnp.int32)])


def _sc_body(src_hbm, dst_hbm, ew_hbm, feat_hbm, src128_hbm,
             acc_out, dh_out,
             acc_sh, hist_s_sh, hist_d_sh,
             sblk, dblk, eblk, sblk1, ones_t, ones_c, norm_s_t, nd_t,
             rows, w_t, gsem, ssem, hsem):
    core = lax.axis_index("c")
    sub = lax.axis_index("s")
    zero16 = jnp.zeros((L,), jnp.float32)
    one16 = jnp.ones((L,), jnp.float32)
    row0 = sub * ROWS_PER_TILE

    def _zrow(r, _):
        for q in range(F // L):
            rows.at[0, r][pl.ds(q * L, L)] = zero16
        return _
    lax.fori_loop(0, CHUNK, _zrow, None)
    for q in range(128 // L):
        ones_t[pl.ds(q * L, L)] = one16
    for q in range(CHUNK // L):
        ones_c[pl.ds(q * L, L)] = one16
    for q in range(ROWS_PER_TILE // L):
        nd_t[pl.ds(q * L, L)] = zero16

    for k in range(ROWS_PER_TILE // ZB):
        pltpu.async_copy(rows.at[0, pl.ds(0, ZB)],
                         acc_sh.at[pl.ds(row0 + k * ZB, ZB)], hsem)
    pltpu.sync_copy(nd_t, hist_s_sh.at[pl.ds(row0, ROWS_PER_TILE)])
    pltpu.sync_copy(nd_t, hist_d_sh.at[pl.ds(row0, ROWS_PER_TILE)])
    for k in range(ROWS_PER_TILE // ZB):
        pltpu.make_async_copy(rows.at[0, pl.ds(0, ZB)],
                              acc_sh.at[pl.ds(row0 + k * ZB, ZB)],
                              hsem).wait()
    plsc.subcore_barrier()

    def _h1(k, _):
        p = k % 2
        base = sub * H1R + k * BLK
        pltpu.sync_copy(src128_hbm.at[pl.ds(base, BLK)], sblk1.at[p])

        @pl.when(k >= 1)
        def _drain():
            for _i in range(BLK):
                pltpu.make_async_copy(ones_t, hist_s_sh.at[sblk1.at[0, 0]],
                                      hsem).wait()

        for i in range(BLK):
            pltpu.async_copy(ones_t, hist_s_sh.at[sblk1.at[p, i]],
                             hsem, add=True)
        return _
    lax.fori_loop(0, H1R // BLK, _h1, None)
    for _i in range(BLK):
        pltpu.make_async_copy(ones_t, hist_s_sh.at[sblk1.at[0, 0]],
                              hsem).wait()
    plsc.subcore_barrier()

    def _norm(ref):
        def _n(g, _):
            d = jnp.maximum(ref[pl.ds(g * L, L)], 1.0)
            ref[pl.ds(g * L, L)] = _rsqrt_newton(d)
            return _
        lax.fori_loop(0, ROWS_PER_TILE // L, _n, None)

    pltpu.sync_copy(hist_s_sh.at[pl.ds(row0, ROWS_PER_TILE)], nd_t)
    _norm(nd_t)
    pltpu.sync_copy(nd_t, hist_s_sh.at[pl.ds(row0, ROWS_PER_TILE)])
    plsc.subcore_barrier()
    pltpu.sync_copy(hist_s_sh, norm_s_t)

    base3 = (core * NS + sub) * CPT

    def _g_issue(pp, r, buf):
        pltpu.async_copy(feat_hbm.at[sblk.at[pp, r]], rows.at[buf],
                         gsem.at[buf])

    def _g_wait(buf):
        pltpu.make_async_copy(feat_hbm.at[sblk.at[0, 0]], rows.at[buf],
                              gsem.at[buf]).wait()

    def _s_issue(pp, r, buf):
        pltpu.async_copy(rows.at[buf], acc_sh.at[dblk.at[pp, r]],
                         ssem.at[buf], add=True)
        pltpu.async_copy(ones_c, hist_d_sh.at[dblk.at[pp, r]],
                         ssem.at[buf], add=True)

    def _s_wait(buf):
        pltpu.make_async_copy(rows.at[buf], acc_sh.at[dblk.at[0, 0]],
                              ssem.at[buf]).wait()
        pltpu.make_async_copy(ones_c, hist_d_sh.at[dblk.at[0, 0]],
                              ssem.at[buf]).wait()

    pltpu.sync_copy(src_hbm.at[pl.ds(base3, BLK)], sblk.at[0])
    pltpu.sync_copy(dst_hbm.at[pl.ds(base3, BLK)], dblk.at[0])
    pltpu.sync_copy(ew_hbm.at[pl.ds(base3, BLK)], eblk.at[0])
    pltpu.sync_copy(src_hbm.at[pl.ds(base3 + BLK, BLK)], sblk.at[1])
    pltpu.sync_copy(dst_hbm.at[pl.ds(base3 + BLK, BLK)], dblk.at[1])
    pltpu.sync_copy(ew_hbm.at[pl.ds(base3 + BLK, BLK)], eblk.at[1])
    for i in range(NBUF - 1):
        _g_issue(0, i, i)

    def _b3(b, _):
        p = b % 2
        for i in range(BLK):
            buf = i % NBUF
            _g_wait(buf)
            for q in range(CHUNK // L):
                sv = sblk.at[p, i][pl.ds(q * L, L)]
                ev = eblk.at[p, i][pl.ds(q * L, L)]
                w_t[pl.ds(q * L, L)] = ev * plsc.load_gather(norm_s_t, [sv])

            def _scale(r2, _c):
                for u in range(2):
                    r = 2 * r2 + u
                    w = _bcast(w_t, r)
                    rv = rows.at[buf, r]
                    for q in range(F // L):
                        rv[pl.ds(q * L, L)] = rv[pl.ds(q * L, L)] * w
                return _c
            lax.fori_loop(0, CHUNK // 2, _scale, None)
            _s_issue(p, i, buf)

            pbuf = (i - 1) % NBUF
            if i == 0:
                @pl.when(b > 0)
                def _w0():
                    _s_wait(pbuf)
            else:
                _s_wait(pbuf)
            if i == 5:
                @pl.when(b + 1 < NBLK)
                def _pref():
                    nb = base3 + (b + 1) * BLK
                    pltpu.sync_copy(src_hbm.at[pl.ds(nb, BLK)],
                                    sblk.at[1 - p])
                    pltpu.sync_copy(dst_hbm.at[pl.ds(nb, BLK)],
                                    dblk.at[1 - p])
                    pltpu.sync_copy(ew_hbm.at[pl.ds(nb, BLK)],
                                    eblk.at[1 - p])
            nxt = b * BLK + i + NBUF - 1

            @pl.when(nxt < CPT)
            def _gi():
                if i + NBUF - 1 < BLK:
                    _g_issue(p, i + NBUF - 1, pbuf)
                else:
                    _g_issue(1 - p, i + NBUF - 1 - BLK, pbuf)
        return _
    lax.fori_loop(0, NBLK, _b3, None)
    _s_wait((CPT - 1) % NBUF)
    plsc.subcore_barrier()

    pltpu.sync_copy(acc_sh.at[pl.ds(row0, ROWS_PER_TILE)],
                    acc_out.at[core, pl.ds(row0, ROWS_PER_TILE)])
    pltpu.sync_copy(hist_d_sh.at[pl.ds(row0, ROWS_PER_TILE)],
                    dh_out.at[core, pl.ds(row0, ROWS_PER_TILE)])


def _sc_aggregate(src_p, dst_p, ew_p, feat_p, src128):
    mesh = plsc.VectorSubcoreMesh(core_axis_name="c", subcore_axis_name="s")
    return pl.kernel(
        _sc_body,
        out_type=[
            jax.ShapeDtypeStruct((NC, N_PAD, F), jnp.float32),
            jax.ShapeDtypeStruct((NC, N_PAD), jnp.float32),
        ],
        mesh=mesh,
        compiler_params=pltpu.CompilerParams(needs_layout_passes=False),
        scratch_types=[
            pltpu.VMEM_SHARED((N_PAD, F), jnp.float32),
            pltpu.VMEM_SHARED((N_PAD,), jnp.float32),
            pltpu.VMEM_SHARED((N_PAD,), jnp.float32),
            pltpu.VMEM((2, BLK, CHUNK), jnp.int32),
            pltpu.VMEM((2, BLK, CHUNK), jnp.int32),
            pltpu.VMEM((2, BLK, CHUNK), jnp.float32),
            pltpu.VMEM((2, BLK, 128), jnp.int32),
            pltpu.VMEM((128,), jnp.float32),
            pltpu.VMEM((CHUNK,), jnp.float32),
            pltpu.VMEM((N_PAD,), jnp.float32),
            pltpu.VMEM((ROWS_PER_TILE,), jnp.float32),
            pltpu.VMEM((NBUF, CHUNK, F), jnp.float32),
            pltpu.VMEM((CHUNK,), jnp.float32),
            pltpu.SemaphoreType.DMA((NBUF,)),
            pltpu.SemaphoreType.DMA((NBUF,)),
            pltpu.SemaphoreType.DMA,
        ],
    )(src_p, dst_p, ew_p, feat_p, src128)


def _tc_body(acc_ref, dh_ref, w_ref, b_ref, out_ref):
    p = acc_ref[0] + acc_ref[1]
    y = jnp.dot(p, w_ref[...], preferred_element_type=jnp.float32)
    d = jnp.maximum(dh_ref[0] + dh_ref[1], 1.0)
    out_ref[...] = y * lax.rsqrt(d) + b_ref[...]


def _tc_matmul(acc, dh, W, b2):
    blk = 1000
    grid = (N_NODES // blk,)
    return pl.pallas_call(
        _tc_body,
        grid=grid,
        in_specs=[
            pl.BlockSpec((NC, blk, F), lambda i: (0, i, 0)),
            pl.BlockSpec((NC, blk, 1), lambda i: (0, i, 0)),
            pl.BlockSpec((F, F), lambda i: (0, 0)),
            pl.BlockSpec((1, F), lambda i: (0, 0)),
        ],
        out_specs=pl.BlockSpec((blk, F), lambda i: (i, 0)),
        out_shape=jax.ShapeDtypeStruct((N_NODES, F), jnp.float32),
    )(acc, dh, W, b2)


@jax.jit
def kernel(feat, edge_index, edge_weight, W, b):
    src = edge_index[0].astype(jnp.int32)
    dst = edge_index[1].astype(jnp.int32)
    e = src.shape[0]
    npad = E_PAD - e
    pad_idx = (jnp.arange(npad, dtype=jnp.int32) % (N_PAD - N_NODES)) + N_NODES
    src_flat = jnp.concatenate([src, pad_idx])
    src_p = src_flat.reshape(E_PAD // CHUNK, CHUNK)
    npad1 = H1ROWS * 128 - E_PAD
    pad1 = (jnp.arange(npad1, dtype=jnp.int32) % (N_PAD - N_NODES)) + N_NODES
    src128 = jnp.concatenate([src_flat, pad1]).reshape(H1ROWS, 128)
    dst_p = jnp.concatenate([dst, pad_idx]).reshape(E_PAD // CHUNK, CHUNK)
    ew_p = jnp.concatenate(
        [edge_weight, jnp.zeros((npad,), jnp.float32)]
    ).reshape(E_PAD // CHUNK, CHUNK)
    feat_p = jnp.pad(feat, ((0, N_PAD - feat.shape[0]), (0, 0)))
    acc, dh = _sc_aggregate(src_p, dst_p, ew_p, feat_p, src128)
    return _tc_matmul(acc, dh.reshape(NC, N_PAD, 1), W, b.reshape(1, F))


# speedup vs baseline: 1.6449x; 1.0004x over previous
import functools

import jax
import jax.numpy as jnp
from jax import lax
from jax.experimental import pallas as pl
from jax.experimental.pallas import tpu as pltpu
from jax.experimental.pallas import tpu_sc as plsc

N_NODES = 10000
N_EDGES = 320000
F = 128

NC = 2
NS = 16
L = 16

N_PAD = 10240
CHUNK = 48
BLK = 8
CPT = 224
NBLK = CPT // BLK
E_PAD = NC * NS * CPT * CHUNK
ROWS_PER_TILE = N_PAD // NS
NBUF = 4
ZB = 40
H1ROWS = 2816
H1R = H1ROWS // NS


def _rsqrt_newton(d):
    s = d
    for _ in range(14):
        s = 0.5 * (s + d / s)
    return 1.0 / s


def _bcast(ref, i):
    return plsc.load_gather(ref, [jnp.full((L,), i, jnp.int32)])


def _sc_body(src_hbm, dst_hbm, ew_hbm, feat_hbm, src128_hbm,
             acc_out, dh_out,
             acc_sh, hist_s_sh, hist_d_sh,
             sblk, dblk, eblk, sblk1, ones_t, ones_c, norm_s_t, nd_t,
             rows, w_t, gsem, ssem, hsem):
    core = lax.axis_index("c")
    sub = lax.axis_index("s")
    zero16 = jnp.zeros((L,), jnp.float32)
    one16 = jnp.ones((L,), jnp.float32)
    row0 = sub * ROWS_PER_TILE

    def _zrow(r, _):
        for q in range(F // L):
            rows.at[0, r][pl.ds(q * L, L)] = zero16
        return _
    lax.fori_loop(0, CHUNK, _zrow, None)
    for q in range(128 // L):
        ones_t[pl.ds(q * L, L)] = one16
    for q in range(CHUNK // L):
        ones_c[pl.ds(q * L, L)] = one16
    for q in range(ROWS_PER_TILE // L):
        nd_t[pl.ds(q * L, L)] = zero16

    for k in range(ROWS_PER_TILE // ZB):
        pltpu.async_copy(rows.at[0, pl.ds(0, ZB)],
                         acc_sh.at[pl.ds(row0 + k * ZB, ZB)], hsem)
    pltpu.sync_copy(nd_t, hist_s_sh.at[pl.ds(row0, ROWS_PER_TILE)])
    pltpu.sync_copy(nd_t, hist_d_sh.at[pl.ds(row0, ROWS_PER_TILE)])
    for k in range(ROWS_PER_TILE // ZB):
        pltpu.make_async_copy(rows.at[0, pl.ds(0, ZB)],
                              acc_sh.at[pl.ds(row0 + k * ZB, ZB)],
                              hsem).wait()
    plsc.subcore_barrier()

    scope1 = jax.named_scope("sc_stage1_hist")
    scope1.__enter__()

    def _h1(k, _):
        p = k % 2
        base = sub * H1R + k * BLK
        pltpu.sync_copy(src128_hbm.at[pl.ds(base, BLK)], sblk1.at[p])

        @pl.when(k >= 1)
        def _drain():
            for _i in range(BLK):
                pltpu.make_async_copy(ones_t, hist_s_sh.at[sblk1.at[0, 0]],
                                      hsem).wait()

        for i in range(BLK):
            pltpu.async_copy(ones_t, hist_s_sh.at[sblk1.at[p, i]],
                             hsem, add=True)
        return _
    lax.fori_loop(0, H1R // BLK, _h1, None)
    for _i in range(BLK):
        pltpu.make_async_copy(ones_t, hist_s_sh.at[sblk1.at[0, 0]],
                              hsem).wait()
    plsc.subcore_barrier()
    scope1.__exit__(None, None, None)

    def _norm(ref):
        def _n(g, _):
            d = jnp.maximum(ref[pl.ds(g * L, L)], 1.0)
            ref[pl.ds(g * L, L)] = _rsqrt_newton(d)
            return _
        lax.fori_loop(0, ROWS_PER_TILE // L, _n, None)

    pltpu.sync_copy(hist_s_sh.at[pl.ds(row0, ROWS_PER_TILE)], nd_t)
    _norm(nd_t)
    pltpu.sync_copy(nd_t, hist_s_sh.at[pl.ds(row0, ROWS_PER_TILE)])
    plsc.subcore_barrier()
    pltpu.sync_copy(hist_s_sh, norm_s_t)

    scope3 = jax.named_scope("sc_stage3_main")
    scope3.__enter__()
    base3 = (core * NS + sub) * CPT

    def _g_issue(pp, r, buf):
        pltpu.async_copy(feat_hbm.at[sblk.at[pp, r]], rows.at[buf],
                         gsem.at[buf])

    def _g_wait(buf):
        pltpu.make_async_copy(feat_hbm.at[sblk.at[0, 0]], rows.at[buf],
                              gsem.at[buf]).wait()

    def _s_issue(pp, r, buf):
        pltpu.async_copy(rows.at[buf], acc_sh.at[dblk.at[pp, r]],
                         ssem.at[buf], add=True)
        pltpu.async_copy(ones_c, hist_d_sh.at[dblk.at[pp, r]],
                         ssem.at[buf], add=True)

    def _s_wait(buf):
        pltpu.make_async_copy(rows.at[buf], acc_sh.at[dblk.at[0, 0]],
                              ssem.at[buf]).wait()
        pltpu.make_async_copy(ones_c, hist_d_sh.at[dblk.at[0, 0]],
                              ssem.at[buf]).wait()

    pltpu.sync_copy(src_hbm.at[pl.ds(base3, BLK)], sblk.at[0])
    pltpu.sync_copy(dst_hbm.at[pl.ds(base3, BLK)], dblk.at[0])
    pltpu.sync_copy(ew_hbm.at[pl.ds(base3, BLK)], eblk.at[0])
    pltpu.sync_copy(src_hbm.at[pl.ds(base3 + BLK, BLK)], sblk.at[1])
    pltpu.sync_copy(dst_hbm.at[pl.ds(base3 + BLK, BLK)], dblk.at[1])
    pltpu.sync_copy(ew_hbm.at[pl.ds(base3 + BLK, BLK)], eblk.at[1])
    for i in range(NBUF - 1):
        _g_issue(0, i, i)

    def _b3(b, _):
        p = b % 2
        for i in range(BLK):
            buf = i % NBUF
            _g_wait(buf)
            for q in range(CHUNK // L):
                sv = sblk.at[p, i][pl.ds(q * L, L)]
                ev = eblk.at[p, i][pl.ds(q * L, L)]
                w_t[pl.ds(q * L, L)] = ev * plsc.load_gather(norm_s_t, [sv])

            def _scale(r2, _c):
                for u in range(2):
                    r = 2 * r2 + u
                    w = _bcast(w_t, r)
                    rv = rows.at[buf, r]
                    for q in range(F // L):
                        rv[pl.ds(q * L, L)] = rv[pl.ds(q * L, L)] * w
                return _c
            lax.fori_loop(0, CHUNK // 2, _scale, None)
            _s_issue(p, i, buf)

            pbuf = (i - 1) % NBUF
            if i == 0:
                @pl.when(b > 0)
                def _w0():
                    _s_wait(pbuf)
            else:
                _s_wait(pbuf)
            if i == 5:
                @pl.when(b + 1 < NBLK)
                def _pref():
                    nb = base3 + (b + 1) * BLK
                    pltpu.sync_copy(src_hbm.at[pl.ds(nb, BLK)],
                                    sblk.at[1 - p])
                    pltpu.sync_copy(dst_hbm.at[pl.ds(nb, BLK)],
                                    dblk.at[1 - p])
                    pltpu.sync_copy(ew_hbm.at[pl.ds(nb, BLK)],
                                    eblk.at[1 - p])
            nxt = b * BLK + i + NBUF - 1

            @pl.when(nxt < CPT)
            def _gi():
                if i + NBUF - 1 < BLK:
                    _g_issue(p, i + NBUF - 1, pbuf)
                else:
                    _g_issue(1 - p, i + NBUF - 1 - BLK, pbuf)
        return _
    lax.fori_loop(0, NBLK, _b3, None)
    _s_wait((CPT - 1) % NBUF)
    plsc.subcore_barrier()
    scope3.__exit__(None, None, None)

    pltpu.sync_copy(acc_sh.at[pl.ds(row0, ROWS_PER_TILE)],
                    acc_out.at[core, pl.ds(row0, ROWS_PER_TILE)])
    pltpu.sync_copy(hist_d_sh.at[pl.ds(row0, ROWS_PER_TILE)],
                    dh_out.at[core, pl.ds(row0, ROWS_PER_TILE)])


def _sc_aggregate(src_p, dst_p, ew_p, feat_p, src128):
    mesh = plsc.VectorSubcoreMesh(core_axis_name="c", subcore_axis_name="s")
    return pl.kernel(
        _sc_body,
        out_type=[
            jax.ShapeDtypeStruct((NC, N_PAD, F), jnp.float32),
            jax.ShapeDtypeStruct((NC, N_PAD), jnp.float32),
        ],
        mesh=mesh,
        compiler_params=pltpu.CompilerParams(needs_layout_passes=False),
        scratch_types=[
            pltpu.VMEM_SHARED((N_PAD, F), jnp.float32),
            pltpu.VMEM_SHARED((N_PAD,), jnp.float32),
            pltpu.VMEM_SHARED((N_PAD,), jnp.float32),
            pltpu.VMEM((2, BLK, CHUNK), jnp.int32),
            pltpu.VMEM((2, BLK, CHUNK), jnp.int32),
            pltpu.VMEM((2, BLK, CHUNK), jnp.float32),
            pltpu.VMEM((2, BLK, 128), jnp.int32),
            pltpu.VMEM((128,), jnp.float32),
            pltpu.VMEM((CHUNK,), jnp.float32),
            pltpu.VMEM((N_PAD,), jnp.float32),
            pltpu.VMEM((ROWS_PER_TILE,), jnp.float32),
            pltpu.VMEM((NBUF, CHUNK, F), jnp.float32),
            pltpu.VMEM((CHUNK,), jnp.float32),
            pltpu.SemaphoreType.DMA((NBUF,)),
            pltpu.SemaphoreType.DMA((NBUF,)),
            pltpu.SemaphoreType.DMA,
        ],
    )(src_p, dst_p, ew_p, feat_p, src128)


def _tc_body(acc_ref, dh_ref, w_ref, b_ref, out_ref):
    p = acc_ref[0] + acc_ref[1]
    y = jnp.dot(p, w_ref[...], preferred_element_type=jnp.float32)
    d = jnp.maximum(dh_ref[0] + dh_ref[1], 1.0)
    out_ref[...] = y * lax.rsqrt(d) + b_ref[...]


def _tc_matmul(acc, dh, W, b2):
    blk = 1000
    grid = (N_NODES // blk,)
    return pl.pallas_call(
        _tc_body,
        grid=grid,
        in_specs=[
            pl.BlockSpec((NC, blk, F), lambda i: (0, i, 0)),
            pl.BlockSpec((NC, blk, 1), lambda i: (0, i, 0)),
            pl.BlockSpec((F, F), lambda i: (0, 0)),
            pl.BlockSpec((1, F), lambda i: (0, 0)),
        ],
        out_specs=pl.BlockSpec((blk, F), lambda i: (i, 0)),
        out_shape=jax.ShapeDtypeStruct((N_NODES, F), jnp.float32),
    )(acc, dh, W, b2)


@jax.jit
def kernel(feat, edge_index, edge_weight, W, b):
    src = edge_index[0].astype(jnp.int32)
    dst = edge_index[1].astype(jnp.int32)
    e = src.shape[0]
    npad = E_PAD - e
    pad_idx = (jnp.arange(npad, dtype=jnp.int32) % (N_PAD - N_NODES)) + N_NODES
    src_flat = jnp.concatenate([src, pad_idx])
    src_p = src_flat.reshape(E_PAD // CHUNK, CHUNK)
    npad1 = H1ROWS * 128 - E_PAD
    pad1 = (jnp.arange(npad1, dtype=jnp.int32) % (N_PAD - N_NODES)) + N_NODES
    src128 = jnp.concatenate([src_flat, pad1]).reshape(H1ROWS, 128)
    dst_p = jnp.concatenate([dst, pad_idx]).reshape(E_PAD // CHUNK, CHUNK)
    ew_p = jnp.concatenate(
        [edge_weight, jnp.zeros((npad,), jnp.float32)]
    ).reshape(E_PAD // CHUNK, CHUNK)
    feat_p = jnp.pad(feat, ((0, N_PAD - feat.shape[0]), (0, 0)))
    acc, dh = _sc_aggregate(src_p, dst_p, ew_p, feat_p, src128)
    return _tc_matmul(acc, dh.reshape(NC, N_PAD, 1), W, b.reshape(1, F))


# speedup vs baseline: 1.6713x; 1.0160x over previous
import functools

import jax
import jax.numpy as jnp
from jax import lax
from jax.experimental import pallas as pl
from jax.experimental.pallas import tpu as pltpu
from jax.experimental.pallas import tpu_sc as plsc

N_NODES = 10000
N_EDGES = 320000
F = 128

NC = 2
NS = 16
L = 16

N_PAD = 10240
CHUNK = 48
BLK = 8
CPT = 224
NBLK = CPT // BLK
E_PAD = NC * NS * CPT * CHUNK
ROWS_PER_TILE = N_PAD // NS
NBUF = 4
ZB = 40
H1ROWS = 2816
H1R = H1ROWS // NS


def _rsqrt_newton(d):
    s = d
    for _ in range(14):
        s = 0.5 * (s + d / s)
    return 1.0 / s


def _bcast(ref, i):
    return plsc.load_gather(ref, [jnp.full((L,), i, jnp.int32)])


def _sc_body(src_hbm, dst_hbm, ew_hbm, feat_hbm, src128_hbm,
             acc_out, dh_out,
             acc_sh, hist_s_sh, hist_d_sh,
             sblk, dblk, eblk, sblk1, ones_t, ones_c, norm_s_t, nd_t,
             rows, w_t, gsem, ssem, hsem):
    core = lax.axis_index("c")
    sub = lax.axis_index("s")
    zero16 = jnp.zeros((L,), jnp.float32)
    one16 = jnp.ones((L,), jnp.float32)
    row0 = sub * ROWS_PER_TILE

    def _zrow(r, _):
        for q in range(F // L):
            rows.at[0, r][pl.ds(q * L, L)] = zero16
        return _
    lax.fori_loop(0, CHUNK, _zrow, None)
    for q in range(128 // L):
        ones_t[pl.ds(q * L, L)] = one16
    for q in range(CHUNK // L):
        ones_c[pl.ds(q * L, L)] = one16
    for q in range(ROWS_PER_TILE // L):
        nd_t[pl.ds(q * L, L)] = zero16

    for k in range(ROWS_PER_TILE // ZB):
        pltpu.async_copy(rows.at[0, pl.ds(0, ZB)],
                         acc_sh.at[pl.ds(row0 + k * ZB, ZB)], hsem)
    pltpu.sync_copy(nd_t, hist_s_sh.at[pl.ds(row0, ROWS_PER_TILE)])
    pltpu.sync_copy(nd_t, hist_d_sh.at[pl.ds(row0, ROWS_PER_TILE)])
    for k in range(ROWS_PER_TILE // ZB):
        pltpu.make_async_copy(rows.at[0, pl.ds(0, ZB)],
                              acc_sh.at[pl.ds(row0 + k * ZB, ZB)],
                              hsem).wait()
    plsc.subcore_barrier()

    def _h1(k, _):
        p = k % 2
        base = sub * H1R + k * BLK
        pltpu.sync_copy(src128_hbm.at[pl.ds(base, BLK)], sblk1.at[p])

        @pl.when(k >= 1)
        def _drain():
            for _i in range(BLK):
                pltpu.make_async_copy(ones_t, hist_s_sh.at[sblk1.at[0, 0]],
                                      hsem).wait()

        for i in range(BLK):
            pltpu.async_copy(ones_t, hist_s_sh.at[sblk1.at[p, i]],
                             hsem, add=True)
        return _
    lax.fori_loop(0, H1R // BLK, _h1, None)
    for _i in range(BLK):
        pltpu.make_async_copy(ones_t, hist_s_sh.at[sblk1.at[0, 0]],
                              hsem).wait()
    plsc.subcore_barrier()

    def _norm(ref):
        def _n(g, _):
            d = jnp.maximum(ref[pl.ds(g * L, L)], 1.0)
            ref[pl.ds(g * L, L)] = _rsqrt_newton(d)
            return _
        lax.fori_loop(0, ROWS_PER_TILE // L, _n, None)

    pltpu.sync_copy(hist_s_sh.at[pl.ds(row0, ROWS_PER_TILE)], nd_t)
    _norm(nd_t)
    pltpu.sync_copy(nd_t, hist_s_sh.at[pl.ds(row0, ROWS_PER_TILE)])
    plsc.subcore_barrier()
    pltpu.sync_copy(hist_s_sh, norm_s_t)

    base3 = (core * NS + sub) * CPT

    def _g_issue(pp, r, buf):
        pltpu.async_copy(feat_hbm.at[sblk.at[pp, r]], rows.at[buf],
                         gsem.at[buf])

    def _g_wait(buf):
        pltpu.make_async_copy(feat_hbm.at[sblk.at[0, 0]], rows.at[buf],
                              gsem.at[buf]).wait()

    def _s_issue(pp, r, buf):
        pltpu.async_copy(rows.at[buf], acc_sh.at[dblk.at[pp, r]],
                         ssem.at[buf], add=True)
        pltpu.async_copy(ones_c, hist_d_sh.at[dblk.at[pp, r]],
                         ssem.at[buf], add=True)

    def _s_wait(buf):
        pltpu.make_async_copy(rows.at[buf], acc_sh.at[dblk.at[0, 0]],
                              ssem.at[buf]).wait()
        pltpu.make_async_copy(ones_c, hist_d_sh.at[dblk.at[0, 0]],
                              ssem.at[buf]).wait()

    pltpu.sync_copy(src_hbm.at[pl.ds(base3, BLK)], sblk.at[0])
    pltpu.sync_copy(dst_hbm.at[pl.ds(base3, BLK)], dblk.at[0])
    pltpu.sync_copy(ew_hbm.at[pl.ds(base3, BLK)], eblk.at[0])
    pltpu.sync_copy(src_hbm.at[pl.ds(base3 + BLK, BLK)], sblk.at[1])
    pltpu.sync_copy(dst_hbm.at[pl.ds(base3 + BLK, BLK)], dblk.at[1])
    pltpu.sync_copy(ew_hbm.at[pl.ds(base3 + BLK, BLK)], eblk.at[1])
    for i in range(NBUF - 1):
        _g_issue(0, i, i)

    def _b3(b, _):
        p = b % 2
        for i in range(BLK):
            buf = i % NBUF
            _g_wait(buf)
            for q in range(CHUNK // L):
                sv = sblk.at[p, i][pl.ds(q * L, L)]
                ev = eblk.at[p, i][pl.ds(q * L, L)]
                w_t[pl.ds(q * L, L)] = ev * plsc.load_gather(norm_s_t, [sv])

            def _scale(r2, _c):
                for u in range(2):
                    r = 2 * r2 + u
                    w = _bcast(w_t, r)
                    rv = rows.at[buf, r]
                    for q in range(F // L):
                        rv[pl.ds(q * L, L)] = rv[pl.ds(q * L, L)] * w
                return _c
            lax.fori_loop(0, CHUNK // 2, _scale, None)
            _s_issue(p, i, buf)

            pbuf = (i - 1) % NBUF
            if i == 0:
                @pl.when(b > 0)
                def _w0():
                    _s_wait(pbuf)
            else:
                _s_wait(pbuf)
            if i == 5:
                @pl.when(b + 1 < NBLK)
                def _pref():
                    nb = base3 + (b + 1) * BLK
                    pltpu.sync_copy(src_hbm.at[pl.ds(nb, BLK)],
                                    sblk.at[1 - p])
                    pltpu.sync_copy(dst_hbm.at[pl.ds(nb, BLK)],
                                    dblk.at[1 - p])
                    pltpu.sync_copy(ew_hbm.at[pl.ds(nb, BLK)],
                                    eblk.at[1 - p])
            nxt = b * BLK + i + NBUF - 1

            @pl.when(nxt < CPT)
            def _gi():
                if i + NBUF - 1 < BLK:
                    _g_issue(p, i + NBUF - 1, pbuf)
                else:
                    _g_issue(1 - p, i + NBUF - 1 - BLK, pbuf)
        return _
    lax.fori_loop(0, NBLK, _b3, None)
    _s_wait((CPT - 1) % NBUF)
    plsc.subcore_barrier()

    pltpu.sync_copy(acc_sh.at[pl.ds(row0, ROWS_PER_TILE)],
                    acc_out.at[core, pl.ds(row0, ROWS_PER_TILE)])
    pltpu.sync_copy(hist_d_sh.at[pl.ds(row0, ROWS_PER_TILE)],
                    dh_out.at[core, pl.ds(row0, ROWS_PER_TILE)])


def _sc_aggregate(src_p, dst_p, ew_p, feat_p, src128):
    mesh = plsc.VectorSubcoreMesh(core_axis_name="c", subcore_axis_name="s")
    return pl.kernel(
        _sc_body,
        out_type=[
            jax.ShapeDtypeStruct((NC, N_PAD, F), jnp.float32),
            jax.ShapeDtypeStruct((NC, N_PAD), jnp.float32),
        ],
        mesh=mesh,
        compiler_params=pltpu.CompilerParams(needs_layout_passes=False),
        scratch_types=[
            pltpu.VMEM_SHARED((N_PAD, F), jnp.float32),
            pltpu.VMEM_SHARED((N_PAD,), jnp.float32),
            pltpu.VMEM_SHARED((N_PAD,), jnp.float32),
            pltpu.VMEM((2, BLK, CHUNK), jnp.int32),
            pltpu.VMEM((2, BLK, CHUNK), jnp.int32),
            pltpu.VMEM((2, BLK, CHUNK), jnp.float32),
            pltpu.VMEM((2, BLK, 128), jnp.int32),
            pltpu.VMEM((128,), jnp.float32),
            pltpu.VMEM((CHUNK,), jnp.float32),
            pltpu.VMEM((N_PAD,), jnp.float32),
            pltpu.VMEM((ROWS_PER_TILE,), jnp.float32),
            pltpu.VMEM((NBUF, CHUNK, F), jnp.float32),
            pltpu.VMEM((CHUNK,), jnp.float32),
            pltpu.SemaphoreType.DMA((NBUF,)),
            pltpu.SemaphoreType.DMA((NBUF,)),
            pltpu.SemaphoreType.DMA,
        ],
    )(src_p, dst_p, ew_p, feat_p, src128)


def _tc_body(acc_ref, dh_ref, w_ref, b_ref, out_ref):
    p = acc_ref[0] + acc_ref[1]
    y = jnp.dot(p, w_ref[...], preferred_element_type=jnp.float32)
    d = jnp.maximum(dh_ref[0] + dh_ref[1], 1.0)
    out_ref[...] = y * lax.rsqrt(d) + b_ref[...]


def _tc_matmul(acc, dh, W, b2):
    blk = 1000
    grid = (N_NODES // blk,)
    return pl.pallas_call(
        _tc_body,
        grid=grid,
        in_specs=[
            pl.BlockSpec((NC, blk, F), lambda i: (0, i, 0)),
            pl.BlockSpec((NC, blk, 1), lambda i: (0, i, 0)),
            pl.BlockSpec((F, F), lambda i: (0, 0)),
            pl.BlockSpec((1, F), lambda i: (0, 0)),
        ],
        out_specs=pl.BlockSpec((blk, F), lambda i: (i, 0)),
        out_shape=jax.ShapeDtypeStruct((N_NODES, F), jnp.float32),
    )(acc, dh, W, b2)


@jax.jit
def kernel(feat, edge_index, edge_weight, W, b):
    src = edge_index[0].astype(jnp.int32)
    dst = edge_index[1].astype(jnp.int32)
    e = src.shape[0]
    npad = E_PAD - e
    pad_idx = (jnp.arange(npad, dtype=jnp.int32) % (N_PAD - N_NODES)) + N_NODES
    pad_src3 = jnp.arange(npad, dtype=jnp.int32) % N_NODES
    src_flat = jnp.concatenate([src, pad_src3])
    src_p = src_flat.reshape(E_PAD // CHUNK, CHUNK)
    npad1 = H1ROWS * 128 - e
    pad1 = (jnp.arange(npad1, dtype=jnp.int32) % (N_PAD - N_NODES)) + N_NODES
    src128 = jnp.concatenate([src, pad1]).reshape(H1ROWS, 128)
    dst_p = jnp.concatenate([dst, pad_idx]).reshape(E_PAD // CHUNK, CHUNK)
    ew_p = jnp.concatenate(
        [edge_weight, jnp.zeros((npad,), jnp.float32)]
    ).reshape(E_PAD // CHUNK, CHUNK)
    acc, dh = _sc_aggregate(src_p, dst_p, ew_p, feat, src128)
    return _tc_matmul(acc, dh.reshape(NC, N_PAD, 1), W, b.reshape(1, F))


# speedup vs baseline: 1.6718x; 1.0003x over previous
import jax
import jax.numpy as jnp
from jax import lax
from jax.experimental import pallas as pl
from jax.experimental.pallas import tpu as pltpu
from jax.experimental.pallas import tpu_sc as plsc

N_NODES = 10000
N_EDGES = 320000
F = 128

NC = 2
NS = 16
L = 16

N_PAD = 10240
CHUNK = 48
BLK = 8
CPT = 224
NBLK = CPT // BLK
E_PAD = NC * NS * CPT * CHUNK
ROWS_PER_TILE = N_PAD // NS
NBUF = 4
ZB = 40
H1ROWS = 2816
H1R = H1ROWS // NS


def _rsqrt_newton(d):
    s = d
    for _ in range(14):
        s = 0.5 * (s + d / s)
    return 1.0 / s


def _bcast(ref, i):
    return plsc.load_gather(ref, [jnp.full((L,), i, jnp.int32)])


def _sc_body(src_hbm, dst_hbm, ew_hbm, feat_hbm, src128_hbm,
             acc_out, dh_out,
             acc_sh, hist_s_sh, hist_d_sh,
             sblk, dblk, eblk, sblk1, ones_t, ones_c, norm_s_t, nd_t,
             rows, w_t, gsem, ssem, hsem):
    core = lax.axis_index("c")
    sub = lax.axis_index("s")
    zero16 = jnp.zeros((L,), jnp.float32)
    one16 = jnp.ones((L,), jnp.float32)
    row0 = sub * ROWS_PER_TILE

    def _zrow(r, _):
        for q in range(F // L):
            rows.at[0, r][pl.ds(q * L, L)] = zero16
        return _
    lax.fori_loop(0, CHUNK, _zrow, None)
    for q in range(128 // L):
        ones_t[pl.ds(q * L, L)] = one16
    for q in range(CHUNK // L):
        ones_c[pl.ds(q * L, L)] = one16
    for q in range(ROWS_PER_TILE // L):
        nd_t[pl.ds(q * L, L)] = zero16

    for k in range(ROWS_PER_TILE // ZB):
        pltpu.async_copy(rows.at[0, pl.ds(0, ZB)],
                         acc_sh.at[pl.ds(row0 + k * ZB, ZB)], hsem)
    pltpu.sync_copy(nd_t, hist_s_sh.at[pl.ds(row0, ROWS_PER_TILE)])
    pltpu.sync_copy(nd_t, hist_d_sh.at[pl.ds(row0, ROWS_PER_TILE)])
    for k in range(ROWS_PER_TILE // ZB):
        pltpu.make_async_copy(rows.at[0, pl.ds(0, ZB)],
                              acc_sh.at[pl.ds(row0 + k * ZB, ZB)],
                              hsem).wait()
    plsc.subcore_barrier()

    def _h1(k, _):
        p = k % 2
        base = sub * H1R + k * BLK
        pltpu.sync_copy(src128_hbm.at[pl.ds(base, BLK)], sblk1.at[p])

        @pl.when(k >= 1)
        def _drain():
            for _i in range(BLK):
                pltpu.make_async_copy(ones_t, hist_s_sh.at[sblk1.at[0, 0]],
                                      hsem).wait()

        for i in range(BLK):
            pltpu.async_copy(ones_t, hist_s_sh.at[sblk1.at[p, i]],
                             hsem, add=True)
        return _
    lax.fori_loop(0, H1R // BLK, _h1, None)
    for _i in range(BLK):
        pltpu.make_async_copy(ones_t, hist_s_sh.at[sblk1.at[0, 0]],
                              hsem).wait()
    plsc.subcore_barrier()

    def _norm(ref):
        def _n(g, _):
            d = jnp.maximum(ref[pl.ds(g * L, L)], 1.0)
            ref[pl.ds(g * L, L)] = _rsqrt_newton(d)
            return _
        lax.fori_loop(0, ROWS_PER_TILE // L, _n, None)

    pltpu.sync_copy(hist_s_sh.at[pl.ds(row0, ROWS_PER_TILE)], nd_t)
    _norm(nd_t)
    pltpu.sync_copy(nd_t, hist_s_sh.at[pl.ds(row0, ROWS_PER_TILE)])
    plsc.subcore_barrier()
    pltpu.sync_copy(hist_s_sh, norm_s_t)

    base3 = (core * NS + sub) * CPT

    def _g_issue(pp, r, buf):
        pltpu.async_copy(feat_hbm.at[sblk.at[pp, r]], rows.at[buf],
                         gsem.at[buf])

    def _g_wait(buf):
        pltpu.make_async_copy(feat_hbm.at[sblk.at[0, 0]], rows.at[buf],
                              gsem.at[buf]).wait()

    def _s_issue(pp, r, buf):
        pltpu.async_copy(rows.at[buf], acc_sh.at[dblk.at[pp, r]],
                         ssem.at[buf], add=True)
        pltpu.async_copy(ones_c, hist_d_sh.at[dblk.at[pp, r]],
                         ssem.at[buf], add=True)

    def _s_wait(buf):
        pltpu.make_async_copy(rows.at[buf], acc_sh.at[dblk.at[0, 0]],
                              ssem.at[buf]).wait()
        pltpu.make_async_copy(ones_c, hist_d_sh.at[dblk.at[0, 0]],
                              ssem.at[buf]).wait()

    pltpu.sync_copy(src_hbm.at[pl.ds(base3, BLK)], sblk.at[0])
    pltpu.sync_copy(dst_hbm.at[pl.ds(base3, BLK)], dblk.at[0])
    pltpu.sync_copy(ew_hbm.at[pl.ds(base3, BLK)], eblk.at[0])
    pltpu.sync_copy(src_hbm.at[pl.ds(base3 + BLK, BLK)], sblk.at[1])
    pltpu.sync_copy(dst_hbm.at[pl.ds(base3 + BLK, BLK)], dblk.at[1])
    pltpu.sync_copy(ew_hbm.at[pl.ds(base3 + BLK, BLK)], eblk.at[1])
    for i in range(NBUF - 1):
        _g_issue(0, i, i)

    def _b3(b, _):
        p = b % 2
        for i in range(BLK):
            buf = i % NBUF
            _g_wait(buf)
            for q in range(CHUNK // L):
                sv = sblk.at[p, i][pl.ds(q * L, L)]
                ev = eblk.at[p, i][pl.ds(q * L, L)]
                w_t[pl.ds(q * L, L)] = ev * plsc.load_gather(norm_s_t, [sv])

            def _scale(r2, _c):
                for u in range(2):
                    r = 2 * r2 + u
                    w = _bcast(w_t, r)
                    rv = rows.at[buf, r]
                    for q in range(F // L):
                        rv[pl.ds(q * L, L)] = rv[pl.ds(q * L, L)] * w
                return _c
            lax.fori_loop(0, CHUNK // 2, _scale, None)
            _s_issue(p, i, buf)

            pbuf = (i - 1) % NBUF
            if i == 0:
                @pl.when(b > 0)
                def _w0():
                    _s_wait(pbuf)
            else:
                _s_wait(pbuf)
            if i == 5:
                @pl.when(b + 1 < NBLK)
                def _pref():
                    nb = base3 + (b + 1) * BLK
                    pltpu.sync_copy(src_hbm.at[pl.ds(nb, BLK)],
                                    sblk.at[1 - p])
                    pltpu.sync_copy(dst_hbm.at[pl.ds(nb, BLK)],
                                    dblk.at[1 - p])
                    pltpu.sync_copy(ew_hbm.at[pl.ds(nb, BLK)],
                                    eblk.at[1 - p])
            nxt = b * BLK + i + NBUF - 1

            @pl.when(nxt < CPT)
            def _gi():
                if i + NBUF - 1 < BLK:
                    _g_issue(p, i + NBUF - 1, pbuf)
                else:
                    _g_issue(1 - p, i + NBUF - 1 - BLK, pbuf)
        return _
    lax.fori_loop(0, NBLK, _b3, None)
    _s_wait((CPT - 1) % NBUF)
    plsc.subcore_barrier()

    pltpu.sync_copy(acc_sh.at[pl.ds(row0, ROWS_PER_TILE)],
                    acc_out.at[core, pl.ds(row0, ROWS_PER_TILE)])
    pltpu.sync_copy(hist_d_sh.at[pl.ds(row0, ROWS_PER_TILE)],
                    dh_out.at[core, pl.ds(row0, ROWS_PER_TILE)])


def _sc_aggregate(src_p, dst_p, ew_p, feat_p, src128):
    mesh = plsc.VectorSubcoreMesh(core_axis_name="c", subcore_axis_name="s")
    return pl.kernel(
        _sc_body,
        out_type=[
            jax.ShapeDtypeStruct((NC, N_PAD, F), jnp.float32),
            jax.ShapeDtypeStruct((NC, N_PAD), jnp.float32),
        ],
        mesh=mesh,
        compiler_params=pltpu.CompilerParams(needs_layout_passes=False),
        scratch_types=[
            pltpu.VMEM_SHARED((N_PAD, F), jnp.float32),
            pltpu.VMEM_SHARED((N_PAD,), jnp.float32),
            pltpu.VMEM_SHARED((N_PAD,), jnp.float32),
            pltpu.VMEM((2, BLK, CHUNK), jnp.int32),
            pltpu.VMEM((2, BLK, CHUNK), jnp.int32),
            pltpu.VMEM((2, BLK, CHUNK), jnp.float32),
            pltpu.VMEM((2, BLK, 128), jnp.int32),
            pltpu.VMEM((128,), jnp.float32),
            pltpu.VMEM((CHUNK,), jnp.float32),
            pltpu.VMEM((N_PAD,), jnp.float32),
            pltpu.VMEM((ROWS_PER_TILE,), jnp.float32),
            pltpu.VMEM((NBUF, CHUNK, F), jnp.float32),
            pltpu.VMEM((CHUNK,), jnp.float32),
            pltpu.SemaphoreType.DMA((NBUF,)),
            pltpu.SemaphoreType.DMA((NBUF,)),
            pltpu.SemaphoreType.DMA,
        ],
    )(src_p, dst_p, ew_p, feat_p, src128)


def _tc_body(acc_ref, dh_ref, w_ref, b_ref, out_ref):
    p = acc_ref[0] + acc_ref[1]
    y = jnp.dot(p, w_ref[...], preferred_element_type=jnp.float32)
    d = jnp.maximum(dh_ref[0] + dh_ref[1], 1.0)
    out_ref[...] = y * lax.rsqrt(d) + b_ref[...]


def _tc_matmul(acc, dh, W, b2):
    blk = 1000
    grid = (N_NODES // blk,)
    return pl.pallas_call(
        _tc_body,
        grid=grid,
        in_specs=[
            pl.BlockSpec((NC, blk, F), lambda i: (0, i, 0)),
            pl.BlockSpec((NC, blk, 1), lambda i: (0, i, 0)),
            pl.BlockSpec((F, F), lambda i: (0, 0)),
            pl.BlockSpec((1, F), lambda i: (0, 0)),
        ],
        out_specs=pl.BlockSpec((blk, F), lambda i: (i, 0)),
        out_shape=jax.ShapeDtypeStruct((N_NODES, F), jnp.float32),
    )(acc, dh, W, b2)


@jax.jit
def kernel(feat, edge_index, edge_weight, W, b):
    src = edge_index[0].astype(jnp.int32)
    dst = edge_index[1].astype(jnp.int32)
    e = src.shape[0]
    npad = E_PAD - e
    pad_idx = (jnp.arange(npad, dtype=jnp.int32) % (N_PAD - N_NODES)) + N_NODES
    pad_src3 = jnp.arange(npad, dtype=jnp.int32) % N_NODES
    src_flat = jnp.concatenate([src, pad_src3])
    src_p = src_flat.reshape(E_PAD // CHUNK, CHUNK)
    npad1 = H1ROWS * 128 - e
    pad1 = (jnp.arange(npad1, dtype=jnp.int32) % (N_PAD - N_NODES)) + N_NODES
    src128 = jnp.concatenate([src, pad1]).reshape(H1ROWS, 128)
    dst_p = jnp.concatenate([dst, pad_idx]).reshape(E_PAD // CHUNK, CHUNK)
    ew_p = jnp.concatenate(
        [edge_weight, jnp.zeros((npad,), jnp.float32)]
    ).reshape(E_PAD // CHUNK, CHUNK)
    acc, dh = _sc_aggregate(src_p, dst_p, ew_p, feat, src128)
    return _tc_matmul(acc, dh.reshape(NC, N_PAD, 1), W, b.reshape(1, F))
